# baseline (speedup 1.0000x reference)
import jax
import jax.numpy as jnp
import numpy as np
from jax import lax
from jax.experimental import pallas as pl
from jax.experimental.pallas import tpu as pltpu
from jax.experimental.pallas import tpu_sc as plsc

NC = 2
NS = 16
NW = NC * NS
K = 128


def _sc_degree(eif, cz, e, n_pad):
    ept = e // NW
    cpt = ept // K
    tail = ept - cpt * K
    zr = n_pad // NS

    def body(ei_hbm, cz_hbm, out_hbm, didx, dtail, ones_v, dacc, isem):
        cid = lax.axis_index("c")
        sid = lax.axis_index("s")
        wid = cid * NS + sid
        base = e + wid * ept
        pltpu.sync_copy(cz_hbm.at[pl.ds(n_pad, K), :], ones_v)
        pltpu.sync_copy(cz_hbm.at[pl.ds(0, zr), :],
                        dacc.at[pl.ds(sid * zr, zr), :])
        pltpu.sync_copy(ei_hbm.at[pl.ds(base, K)], didx.at[0])
        plsc.subcore_barrier()

        def step(i, carry):
            b = i & 1
            nb = (i + 1) & 1

            @pl.when(i + 1 < cpt)
            def _():
                pltpu.async_copy(ei_hbm.at[pl.ds(base + (i + 1) * K, K)],
                                 didx.at[nb], isem)

            pltpu.sync_copy(ones_v, dacc.at[didx.at[b]], add=True)

            @pl.when(i + 1 < cpt)
            def _():
                pltpu.make_async_copy(ei_hbm.at[pl.ds(0, K)], didx.at[nb],
                                      isem).wait()

            return carry

        lax.fori_loop(0, cpt, step, 0)
        if tail:
            pltpu.sync_copy(ei_hbm.at[pl.ds(base + cpt * K, tail)], dtail)
            pltpu.sync_copy(ones_v.at[pl.ds(0, tail), :], dacc.at[dtail],
                            add=True)
        plsc.subcore_barrier()
        pltpu.sync_copy(dacc.at[pl.ds(sid * zr, zr), :],
                        out_hbm.at[cid, pl.ds(sid * zr, zr), :])

    fn = pl.kernel(
        body,
        out_type=jax.ShapeDtypeStruct((NC, n_pad, 8), jnp.float32),
        mesh=plsc.VectorSubcoreMesh(core_axis_name="c", subcore_axis_name="s"),
        scratch_types=[
            pltpu.VMEM((2, K), jnp.int32),
            pltpu.VMEM((max(tail, 8),), jnp.int32),
            pltpu.VMEM((K, 8), jnp.float32),
            pltpu.VMEM_SHARED((n_pad, 8), jnp.float32),
            pltpu.SemaphoreType.DMA,
        ],
    )
    return fn(eif, cz)


def _sc_spmm(y, eif, e, n_pad):
    f = y.shape[1]
    ept = e // NW
    cpt = ept // K
    tail = ept - cpt * K
    zbr = 64
    zr = n_pad // NS

    def body(y_hbm, ei_hbm, out_hbm, sidx, didx, stail, dtail, rows, rtail,
             zbuf, acc, gsem, isem, ssem):
        cid = lax.axis_index("c")
        sid = lax.axis_index("s")
        wid = cid * NS + sid
        bs = wid * ept
        bd = e + wid * ept

        def zb(r, carry):
            for j in range(f // 16):
                zbuf[r, pl.ds(j * 16, 16)] = jnp.zeros((16,), jnp.float32)
            return carry

        lax.fori_loop(0, zbr, zb, 0)
        pltpu.sync_copy(ei_hbm.at[pl.ds(bs, K)], sidx.at[0])
        pltpu.sync_copy(ei_hbm.at[pl.ds(bd, K)], didx.at[0])
        for k in range(zr // zbr):
            pltpu.sync_copy(zbuf, acc.at[pl.ds(sid * zr + k * zbr, zbr), :])
        plsc.subcore_barrier()

        pltpu.async_copy(y_hbm.at[sidx.at[0]], rows.at[0], gsem)
        pltpu.async_copy(ei_hbm.at[pl.ds(bs + K, K)], sidx.at[1], isem)
        pltpu.async_copy(ei_hbm.at[pl.ds(bd + K, K)], didx.at[1], isem)

        def step(i, carry):
            b = i & 1
            nb = (i + 1) & 1
            pltpu.make_async_copy(y_hbm.at[sidx.at[b]], rows.at[b],
                                  gsem).wait()
            pltpu.async_copy(rows.at[b], acc.at[didx.at[i & 3]], ssem,
                             add=True)

            @pl.when(i + 1 < cpt)
            def _():
                @pl.when(i >= 1)
                def _():
                    pltpu.make_async_copy(y_hbm.at[sidx.at[0]], rows.at[nb],
                                          ssem).wait()

                pltpu.make_async_copy(ei_hbm.at[pl.ds(0, K)], sidx.at[nb],
                                      isem).wait()
                pltpu.make_async_copy(ei_hbm.at[pl.ds(0, K)],
                                      didx.at[(i + 1) & 3], isem).wait()
                pltpu.async_copy(y_hbm.at[sidx.at[nb]], rows.at[nb], gsem)

            @pl.when(i + 2 < cpt)
            def _():
                pltpu.async_copy(ei_hbm.at[pl.ds(bs + (i + 2) * K, K)],
                                 sidx.at[b], isem)
                pltpu.async_copy(ei_hbm.at[pl.ds(bd + (i + 2) * K, K)],
                                 didx.at[(i + 2) & 3], isem)

            return carry

        lax.fori_loop(0, cpt, step, 0)
        pltpu.make_async_copy(y_hbm.at[sidx.at[0]], rows.at[0], ssem).wait()
        pltpu.make_async_copy(y_hbm.at[sidx.at[0]], rows.at[1], ssem).wait()
        if tail:
            pltpu.sync_copy(ei_hbm.at[pl.ds(bs + cpt * K, tail)], stail)
            pltpu.sync_copy(ei_hbm.at[pl.ds(bd + cpt * K, tail)], dtail)
            pltpu.async_copy(y_hbm.at[stail], rtail, gsem).wait()
            pltpu.sync_copy(rtail, acc.at[dtail], add=True)
        plsc.subcore_barrier()
        pltpu.sync_copy(acc.at[pl.ds(sid * zr, zr), :],
                        out_hbm.at[cid, pl.ds(sid * zr, zr), :])

    fn = pl.kernel(
        body,
        out_type=jax.ShapeDtypeStruct((NC, n_pad, f), jnp.float32),
        mesh=plsc.VectorSubcoreMesh(core_axis_name="c", subcore_axis_name="s"),
        scratch_types=[
            pltpu.VMEM((2, K), jnp.int32),
            pltpu.VMEM((4, K), jnp.int32),
            pltpu.VMEM((max(tail, 8),), jnp.int32),
            pltpu.VMEM((max(tail, 8),), jnp.int32),
            pltpu.VMEM((2, K, f), jnp.float32),
            pltpu.VMEM((max(tail, 8), f), jnp.float32),
            pltpu.VMEM((zbr, f), jnp.float32),
            pltpu.VMEM_SHARED((n_pad, f), jnp.float32),
            pltpu.SemaphoreType.DMA,
            pltpu.SemaphoreType.DMA,
            pltpu.SemaphoreType.DMA,
        ],
    )
    return fn(y, eif)


def _tc_matmul(x, em, w0, b0, w1):
    n = x.shape[0]

    def body(x_r, em_r, w0_r, b0_r, w1_r, out_r):
        nw0 = jnp.maximum(w0_r[...] @ em_r[...] + b0_r[...], 0.0)
        m1 = nw0 @ w1_r[...]
        out_r[...] = x_r[...] @ m1

    return pl.pallas_call(
        body,
        out_shape=jax.ShapeDtypeStruct((n, x.shape[1]), jnp.float32),
    )(x, em, w0, b0, w1)


def _tc_scale(xm1, degp):
    n = xm1.shape[0]

    def body(xm_r, deg_r, y1_r, dinv_r):
        dsum = deg_r[0, :n, 0:1] + deg_r[1, :n, 0:1] + 1.0
        dinv = lax.rsqrt(dsum)
        y1_r[...] = xm_r[...] * dinv
        dinv_r[...] = dinv

    return pl.pallas_call(
        body,
        out_shape=[
            jax.ShapeDtypeStruct((n, xm1.shape[1]), jnp.float32),
            jax.ShapeDtypeStruct((n, 1), jnp.float32),
        ],
    )(xm1, degp)


def _tc_mid(z, y, dinv, b, w_next):
    n, f = y.shape

    def body(z_r, y_r, dinv_r, b_r, w_r, out_r):
        ztot = z_r[0, :n] + z_r[1, :n] + y_r[...]
        h = jnp.maximum(ztot * dinv_r[...] + b_r[...], 0.0)
        out_r[...] = (h @ w_r[...]) * dinv_r[...]

    return pl.pallas_call(
        body,
        out_shape=jax.ShapeDtypeStruct((n, f), jnp.float32),
    )(z, y, dinv, b, w_next)


def _tc_post(z, y, dinv, b, ltw, ltb):
    n = y.shape[0]
    c = ltw.shape[0]

    def body(z_r, y_r, dinv_r, b_r, w_r, ltb_r, out_r):
        ztot = z_r[0, :n] + z_r[1, :n] + y_r[...]
        h = jnp.maximum(ztot * dinv_r[...] + b_r[...], 0.0)
        logits = lax.dot_general(h, w_r[...], (((1,), (1,)), ((), ())),
                                 preferred_element_type=jnp.float32)
        logits = logits + ltb_r[...]
        m = jnp.max(logits, axis=1, keepdims=True)
        lse = jnp.log(jnp.sum(jnp.exp(logits - m), axis=1, keepdims=True)) + m
        out_r[...] = logits - lse

    return pl.pallas_call(
        body,
        out_shape=jax.ShapeDtypeStruct((n, c), jnp.float32),
    )(z, y, dinv, b, ltw, ltb)


def kernel(x, edge_index, E_meta, w0, b0, conv1_W, conv1_b, conv2_W, conv2_b,
           lt1_W, lt1_b):
    n, f = x.shape
    e = edge_index.shape[1]

    nblk = NS * 64
    n_pad = ((n + nblk - 1) // nblk) * nblk

    eif = edge_index.astype(jnp.int32).reshape(-1)

    cz_np = np.zeros((n_pad + K, 8), np.float32)
    cz_np[n_pad:] = 1.0
    cz = jnp.asarray(cz_np)

    degp = _sc_degree(eif, cz, e, n_pad)
    xm1 = _tc_matmul(x, E_meta, w0, b0, conv1_W)
    y1, dinv = _tc_scale(xm1, degp)
    z1 = _sc_spmm(y1, eif, e, n_pad)
    y2 = _tc_mid(z1, y1, dinv, conv1_b.reshape(1, -1), conv2_W)
    z2 = _sc_spmm(y2, eif, e, n_pad)
    out = _tc_post(z2, y2, dinv, conv2_b.reshape(1, -1), lt1_W,
                   lt1_b.reshape(1, -1))
    return out


# speedup vs baseline: 1.0824x; 1.0824x over previous
import jax
import jax.numpy as jnp
import numpy as np
from jax import lax
from jax.experimental import pallas as pl
from jax.experimental.pallas import tpu as pltpu
from jax.experimental.pallas import tpu_sc as plsc

NC = 2
NS = 16
NW = NC * NS
K = 128


def _sc_degree(eif, cz, e, n_pad):
    ept = e // NW
    cpt = ept // K
    tail = ept - cpt * K
    zr = n_pad // NS

    def body(ei_hbm, cz_hbm, out_hbm, didx, ones_v, dacc):
        cid = lax.axis_index("c")
        sid = lax.axis_index("s")
        wid = cid * NS + sid
        base = e + wid * ept
        pltpu.sync_copy(cz_hbm.at[pl.ds(n_pad, K), :], ones_v)
        pltpu.sync_copy(ei_hbm.at[pl.ds(base, ept)], didx)
        pltpu.sync_copy(cz_hbm.at[pl.ds(0, zr), :],
                        dacc.at[pl.ds(sid * zr, zr), :])
        plsc.subcore_barrier()

        def step(i, carry):
            pltpu.sync_copy(ones_v, dacc.at[didx.at[pl.ds(i * K, K)]],
                            add=True)
            return carry

        lax.fori_loop(0, cpt, step, 0)
        if tail:
            pltpu.sync_copy(ones_v.at[pl.ds(0, tail), :],
                            dacc.at[didx.at[pl.ds(cpt * K, tail)]], add=True)
        plsc.subcore_barrier()
        pltpu.sync_copy(dacc.at[pl.ds(sid * zr, zr), :],
                        out_hbm.at[cid, pl.ds(sid * zr, zr), :])

    fn = pl.kernel(
        body,
        out_type=jax.ShapeDtypeStruct((NC, n_pad, 8), jnp.float32),
        mesh=plsc.VectorSubcoreMesh(core_axis_name="c", subcore_axis_name="s"),
        scratch_types=[
            pltpu.VMEM((ept,), jnp.int32),
            pltpu.VMEM((K, 8), jnp.float32),
            pltpu.VMEM_SHARED((n_pad, 8), jnp.float32),
        ],
    )
    return fn(eif, cz)


def _sc_spmm(y, eif, e, n_pad):
    f = y.shape[1]
    ept = e // NW
    cpt = ept // K
    tail = ept - cpt * K
    zbr = 64
    zr = n_pad // NS

    def body(y_hbm, ei_hbm, out_hbm, sidx, didx, stail, dtail, rows, rtail,
             zbuf, acc, gsem, isem, ssem):
        cid = lax.axis_index("c")
        sid = lax.axis_index("s")
        wid = cid * NS + sid
        bs = wid * ept
        bd = e + wid * ept

        def zb(r, carry):
            for j in range(f // 16):
                zbuf[r, pl.ds(j * 16, 16)] = jnp.zeros((16,), jnp.float32)
            return carry

        lax.fori_loop(0, zbr, zb, 0)
        pltpu.sync_copy(ei_hbm.at[pl.ds(bs, K)], sidx.at[0])
        pltpu.sync_copy(ei_hbm.at[pl.ds(bd, K)], didx.at[0])
        for k in range(zr // zbr):
            pltpu.sync_copy(zbuf, acc.at[pl.ds(sid * zr + k * zbr, zbr), :])
        plsc.subcore_barrier()

        pltpu.async_copy(y_hbm.at[sidx.at[0]], rows.at[0], gsem)
        pltpu.async_copy(ei_hbm.at[pl.ds(bs + K, K)], sidx.at[1], isem)
        pltpu.async_copy(ei_hbm.at[pl.ds(bd + K, K)], didx.at[1], isem)

        def step(i, carry):
            b = i & 1
            nb = (i + 1) & 1
            pltpu.make_async_copy(y_hbm.at[sidx.at[b]], rows.at[b],
                                  gsem).wait()
            pltpu.async_copy(rows.at[b], acc.at[didx.at[i & 3]], ssem,
                             add=True)

            @pl.when(i + 1 < cpt)
            def _():
                @pl.when(i >= 1)
                def _():
                    pltpu.make_async_copy(y_hbm.at[sidx.at[0]], rows.at[nb],
                                          ssem).wait()

                pltpu.make_async_copy(ei_hbm.at[pl.ds(0, K)], sidx.at[nb],
                                      isem).wait()
                pltpu.make_async_copy(ei_hbm.at[pl.ds(0, K)],
                                      didx.at[(i + 1) & 3], isem).wait()
                pltpu.async_copy(y_hbm.at[sidx.at[nb]], rows.at[nb], gsem)

            @pl.when(i + 2 < cpt)
            def _():
                pltpu.async_copy(ei_hbm.at[pl.ds(bs + (i + 2) * K, K)],
                                 sidx.at[b], isem)
                pltpu.async_copy(ei_hbm.at[pl.ds(bd + (i + 2) * K, K)],
                                 didx.at[(i + 2) & 3], isem)

            return carry

        lax.fori_loop(0, cpt, step, 0)
        pltpu.make_async_copy(y_hbm.at[sidx.at[0]], rows.at[0], ssem).wait()
        pltpu.make_async_copy(y_hbm.at[sidx.at[0]], rows.at[1], ssem).wait()
        if tail:
            pltpu.sync_copy(ei_hbm.at[pl.ds(bs + cpt * K, tail)], stail)
            pltpu.sync_copy(ei_hbm.at[pl.ds(bd + cpt * K, tail)], dtail)
            pltpu.async_copy(y_hbm.at[stail], rtail, gsem).wait()
            pltpu.sync_copy(rtail, acc.at[dtail], add=True)
        plsc.subcore_barrier()
        pltpu.sync_copy(acc.at[pl.ds(sid * zr, zr), :],
                        out_hbm.at[cid, pl.ds(sid * zr, zr), :])

    fn = pl.kernel(
        body,
        out_type=jax.ShapeDtypeStruct((NC, n_pad, f), jnp.float32),
        mesh=plsc.VectorSubcoreMesh(core_axis_name="c", subcore_axis_name="s"),
        scratch_types=[
            pltpu.VMEM((2, K), jnp.int32),
            pltpu.VMEM((4, K), jnp.int32),
            pltpu.VMEM((max(tail, 8),), jnp.int32),
            pltpu.VMEM((max(tail, 8),), jnp.int32),
            pltpu.VMEM((2, K, f), jnp.float32),
            pltpu.VMEM((max(tail, 8), f), jnp.float32),
            pltpu.VMEM((zbr, f), jnp.float32),
            pltpu.VMEM_SHARED((n_pad, f), jnp.float32),
            pltpu.SemaphoreType.DMA,
            pltpu.SemaphoreType.DMA,
            pltpu.SemaphoreType.DMA,
        ],
    )
    return fn(y, eif)


def _tc_matmul(x, em, w0, b0, w1):
    n = x.shape[0]

    def body(x_r, em_r, w0_r, b0_r, w1_r, out_r):
        nw0 = jnp.maximum(w0_r[...] @ em_r[...] + b0_r[...], 0.0)
        m1 = nw0 @ w1_r[...]
        out_r[...] = x_r[...] @ m1

    return pl.pallas_call(
        body,
        out_shape=jax.ShapeDtypeStruct((n, x.shape[1]), jnp.float32),
    )(x, em, w0, b0, w1)


def _tc_scale(xm1, degp):
    n = xm1.shape[0]

    def body(xm_r, deg_r, y1_r, dinv_r):
        dsum = deg_r[0, :n, 0:1] + deg_r[1, :n, 0:1] + 1.0
        dinv = lax.rsqrt(dsum)
        y1_r[...] = xm_r[...] * dinv
        dinv_r[...] = dinv

    return pl.pallas_call(
        body,
        out_shape=[
            jax.ShapeDtypeStruct((n, xm1.shape[1]), jnp.float32),
            jax.ShapeDtypeStruct((n, 1), jnp.float32),
        ],
    )(xm1, degp)


def _tc_mid(z, y, dinv, b, w_next):
    n, f = y.shape

    def body(z_r, y_r, dinv_r, b_r, w_r, out_r):
        ztot = z_r[0, :n] + z_r[1, :n] + y_r[...]
        h = jnp.maximum(ztot * dinv_r[...] + b_r[...], 0.0)
        out_r[...] = (h @ w_r[...]) * dinv_r[...]

    return pl.pallas_call(
        body,
        out_shape=jax.ShapeDtypeStruct((n, f), jnp.float32),
    )(z, y, dinv, b, w_next)


def _tc_post(z, y, dinv, b, ltw, ltb):
    n = y.shape[0]
    c = ltw.shape[0]

    def body(z_r, y_r, dinv_r, b_r, w_r, ltb_r, out_r):
        ztot = z_r[0, :n] + z_r[1, :n] + y_r[...]
        h = jnp.maximum(ztot * dinv_r[...] + b_r[...], 0.0)
        logits = lax.dot_general(h, w_r[...], (((1,), (1,)), ((), ())),
                                 preferred_element_type=jnp.float32)
        logits = logits + ltb_r[...]
        m = jnp.max(logits, axis=1, keepdims=True)
        lse = jnp.log(jnp.sum(jnp.exp(logits - m), axis=1, keepdims=True)) + m
        out_r[...] = logits - lse

    return pl.pallas_call(
        body,
        out_shape=jax.ShapeDtypeStruct((n, c), jnp.float32),
    )(z, y, dinv, b, ltw, ltb)


def kernel(x, edge_index, E_meta, w0, b0, conv1_W, conv1_b, conv2_W, conv2_b,
           lt1_W, lt1_b):
    n, f = x.shape
    e = edge_index.shape[1]

    nblk = NS * 64
    n_pad = ((n + nblk - 1) // nblk) * nblk

    eif = edge_index.astype(jnp.int32).reshape(-1)

    cz_np = np.zeros((n_pad + K, 8), np.float32)
    cz_np[n_pad:] = 1.0
    cz = jnp.asarray(cz_np)

    degp = _sc_degree(eif, cz, e, n_pad)
    xm1 = _tc_matmul(x, E_meta, w0, b0, conv1_W)
    y1, dinv = _tc_scale(xm1, degp)
    z1 = _sc_spmm(y1, eif, e, n_pad)
    y2 = _tc_mid(z1, y1, dinv, conv1_b.reshape(1, -1), conv2_W)
    z2 = _sc_spmm(y2, eif, e, n_pad)
    out = _tc_post(z2, y2, dinv, conv2_b.reshape(1, -1), lt1_W,
                   lt1_b.reshape(1, -1))
    return out


# speedup vs baseline: 1.2784x; 1.1810x over previous
import jax
import jax.numpy as jnp
import numpy as np
from jax import lax
from jax.experimental import pallas as pl
from jax.experimental.pallas import tpu as pltpu
from jax.experimental.pallas import tpu_sc as plsc

NC = 2
NS = 16
NW = NC * NS
K = 128


def _sc_degree(eif, cz, e, n_pad):
    ept = e // NW
    cpt = ept // K
    tail = ept - cpt * K
    zr = n_pad // NS

    def body(ei_hbm, cz_hbm, out_hbm, didx, ones_v, dacc):
        cid = lax.axis_index("c")
        sid = lax.axis_index("s")
        wid = cid * NS + sid
        base = e + wid * ept
        pltpu.sync_copy(cz_hbm.at[pl.ds(n_pad, K), :], ones_v)
        pltpu.sync_copy(ei_hbm.at[pl.ds(base, ept)], didx)
        pltpu.sync_copy(cz_hbm.at[pl.ds(0, zr), :],
                        dacc.at[pl.ds(sid * zr, zr), :])
        plsc.subcore_barrier()

        def step(i, carry):
            pltpu.sync_copy(ones_v, dacc.at[didx.at[pl.ds(i * K, K)]],
                            add=True)
            return carry

        lax.fori_loop(0, cpt, step, 0)
        if tail:
            pltpu.sync_copy(ones_v.at[pl.ds(0, tail), :],
                            dacc.at[didx.at[pl.ds(cpt * K, tail)]], add=True)
        plsc.subcore_barrier()
        pltpu.sync_copy(dacc.at[pl.ds(sid * zr, zr), :],
                        out_hbm.at[cid, pl.ds(sid * zr, zr), :])

    fn = pl.kernel(
        body,
        out_type=jax.ShapeDtypeStruct((NC, n_pad, 8), jnp.float32),
        mesh=plsc.VectorSubcoreMesh(core_axis_name="c", subcore_axis_name="s"),
        scratch_types=[
            pltpu.VMEM((ept,), jnp.int32),
            pltpu.VMEM((K, 8), jnp.float32),
            pltpu.VMEM_SHARED((n_pad, 8), jnp.float32),
        ],
    )
    return fn(eif, cz)


def _sc_spmm(y, eif, e, n_pad):
    f = y.shape[1]
    KS = 64
    ept = e // NW
    cpt = ept // KS
    tail = ept - cpt * KS
    zbr = 64
    zr = n_pad // NS

    def body(y_hbm, ei_hbm, out_hbm, sidx, didx, stail, dtail, rows, rtail,
             zbuf, acc, gsem, isem, ssem):
        cid = lax.axis_index("c")
        sid = lax.axis_index("s")
        wid = cid * NS + sid
        bs = wid * ept
        bd = e + wid * ept

        def zb(r, carry):
            for j in range(f // 16):
                zbuf[r, pl.ds(j * 16, 16)] = jnp.zeros((16,), jnp.float32)
            return carry

        lax.fori_loop(0, zbr, zb, 0)
        for j in range(3):
            pltpu.sync_copy(ei_hbm.at[pl.ds(bs + j * KS, KS)], sidx.at[j])
            pltpu.sync_copy(ei_hbm.at[pl.ds(bd + j * KS, KS)], didx.at[j])
        for k in range(zr // zbr):
            pltpu.sync_copy(zbuf, acc.at[pl.ds(sid * zr + k * zbr, zbr), :])
        plsc.subcore_barrier()

        for j in range(3):
            pltpu.async_copy(y_hbm.at[sidx.at[j]], rows.at[j], gsem)
        pltpu.async_copy(ei_hbm.at[pl.ds(bs + 3 * KS, KS)], sidx.at[3], isem)
        pltpu.async_copy(ei_hbm.at[pl.ds(bd + 3 * KS, KS)], didx.at[3], isem)

        def step(i, carry):
            b = i & 3
            pltpu.make_async_copy(y_hbm.at[sidx.at[b]], rows.at[b],
                                  gsem).wait()
            pltpu.async_copy(rows.at[b], acc.at[didx.at[i & 7]], ssem,
                             add=True)

            @pl.when(i + 3 < cpt)
            def _():
                @pl.when(i >= 1)
                def _():
                    pltpu.make_async_copy(y_hbm.at[sidx.at[0]], rows.at[b],
                                          ssem).wait()

                pltpu.make_async_copy(ei_hbm.at[pl.ds(0, KS)],
                                      sidx.at[(i + 3) & 3], isem).wait()
                pltpu.make_async_copy(ei_hbm.at[pl.ds(0, KS)],
                                      didx.at[(i + 3) & 7], isem).wait()
                pltpu.async_copy(y_hbm.at[sidx.at[(i + 3) & 3]],
                                 rows.at[(i + 3) & 3], gsem)

            @pl.when(i + 4 < cpt)
            def _():
                pltpu.async_copy(ei_hbm.at[pl.ds(bs + (i + 4) * KS, KS)],
                                 sidx.at[b], isem)
                pltpu.async_copy(ei_hbm.at[pl.ds(bd + (i + 4) * KS, KS)],
                                 didx.at[(i + 4) & 7], isem)

            return carry

        lax.fori_loop(0, cpt, step, 0)
        for j in range(4):
            pltpu.make_async_copy(y_hbm.at[sidx.at[0]], rows.at[j], ssem).wait()
        if tail:
            pltpu.sync_copy(ei_hbm.at[pl.ds(bs + cpt * KS, tail)], stail)
            pltpu.sync_copy(ei_hbm.at[pl.ds(bd + cpt * KS, tail)], dtail)
            pltpu.async_copy(y_hbm.at[stail], rtail, gsem).wait()
            pltpu.sync_copy(rtail, acc.at[dtail], add=True)
        plsc.subcore_barrier()
        pltpu.sync_copy(acc.at[pl.ds(sid * zr, zr), :],
                        out_hbm.at[cid, pl.ds(sid * zr, zr), :])

    fn = pl.kernel(
        body,
        out_type=jax.ShapeDtypeStruct((NC, n_pad, f), jnp.float32),
        mesh=plsc.VectorSubcoreMesh(core_axis_name="c", subcore_axis_name="s"),
        scratch_types=[
            pltpu.VMEM((4, KS), jnp.int32),
            pltpu.VMEM((8, KS), jnp.int32),
            pltpu.VMEM((max(tail, 8),), jnp.int32),
            pltpu.VMEM((max(tail, 8),), jnp.int32),
            pltpu.VMEM((4, KS, f), jnp.float32),
            pltpu.VMEM((max(tail, 8), f), jnp.float32),
            pltpu.VMEM((zbr, f), jnp.float32),
            pltpu.VMEM_SHARED((n_pad, f), jnp.float32),
            pltpu.SemaphoreType.DMA,
            pltpu.SemaphoreType.DMA,
            pltpu.SemaphoreType.DMA,
        ],
    )
    return fn(y, eif)


def _tc_matmul(x, em, w0, b0, w1):
    n = x.shape[0]

    def body(x_r, em_r, w0_r, b0_r, w1_r, out_r):
        nw0 = jnp.maximum(w0_r[...] @ em_r[...] + b0_r[...], 0.0)
        m1 = nw0 @ w1_r[...]
        out_r[...] = x_r[...] @ m1

    return pl.pallas_call(
        body,
        out_shape=jax.ShapeDtypeStruct((n, x.shape[1]), jnp.float32),
    )(x, em, w0, b0, w1)


def _tc_scale(xm1, degp):
    n = xm1.shape[0]

    def body(xm_r, deg_r, y1_r, dinv_r):
        dsum = deg_r[0, :n, 0:1] + deg_r[1, :n, 0:1] + 1.0
        dinv = lax.rsqrt(dsum)
        y1_r[...] = xm_r[...] * dinv
        dinv_r[...] = dinv

    return pl.pallas_call(
        body,
        out_shape=[
            jax.ShapeDtypeStruct((n, xm1.shape[1]), jnp.float32),
            jax.ShapeDtypeStruct((n, 1), jnp.float32),
        ],
    )(xm1, degp)


def _tc_mid(z, y, dinv, b, w_next):
    n, f = y.shape

    def body(z_r, y_r, dinv_r, b_r, w_r, out_r):
        ztot = z_r[0, :n] + z_r[1, :n] + y_r[...]
        h = jnp.maximum(ztot * dinv_r[...] + b_r[...], 0.0)
        out_r[...] = (h @ w_r[...]) * dinv_r[...]

    return pl.pallas_call(
        body,
        out_shape=jax.ShapeDtypeStruct((n, f), jnp.float32),
    )(z, y, dinv, b, w_next)


def _tc_post(z, y, dinv, b, ltw, ltb):
    n = y.shape[0]
    c = ltw.shape[0]

    def body(z_r, y_r, dinv_r, b_r, w_r, ltb_r, out_r):
        ztot = z_r[0, :n] + z_r[1, :n] + y_r[...]
        h = jnp.maximum(ztot * dinv_r[...] + b_r[...], 0.0)
        logits = lax.dot_general(h, w_r[...], (((1,), (1,)), ((), ())),
                                 preferred_element_type=jnp.float32)
        logits = logits + ltb_r[...]
        m = jnp.max(logits, axis=1, keepdims=True)
        lse = jnp.log(jnp.sum(jnp.exp(logits - m), axis=1, keepdims=True)) + m
        out_r[...] = logits - lse

    return pl.pallas_call(
        body,
        out_shape=jax.ShapeDtypeStruct((n, c), jnp.float32),
    )(z, y, dinv, b, ltw, ltb)


def kernel(x, edge_index, E_meta, w0, b0, conv1_W, conv1_b, conv2_W, conv2_b,
           lt1_W, lt1_b):
    n, f = x.shape
    e = edge_index.shape[1]

    nblk = NS * 64
    n_pad = ((n + nblk - 1) // nblk) * nblk

    eif = edge_index.astype(jnp.int32).reshape(-1)

    cz_np = np.zeros((n_pad + K, 8), np.float32)
    cz_np[n_pad:] = 1.0
    cz = jnp.asarray(cz_np)

    degp = _sc_degree(eif, cz, e, n_pad)
    xm1 = _tc_matmul(x, E_meta, w0, b0, conv1_W)
    y1, dinv = _tc_scale(xm1, degp)
    z1 = _sc_spmm(y1, eif, e, n_pad)
    y2 = _tc_mid(z1, y1, dinv, conv1_b.reshape(1, -1), conv2_W)
    z2 = _sc_spmm(y2, eif, e, n_pad)
    out = _tc_post(z2, y2, dinv, conv2_b.reshape(1, -1), lt1_W,
                   lt1_b.reshape(1, -1))
    return out
